# Initial kernel scaffold; baseline (speedup 1.0000x reference)
import functools

import jax
import jax.numpy as jnp
from jax import lax
from jax.experimental import pallas as pl
from jax.experimental.pallas import tpu as pltpu
from jax.experimental.pallas import tpu_sc as plsc

BATCH = 16384
HIST = 200
EMBED = 2
N_TOTAL = BATCH * HIST
NC, NS = 2, 16
NW = NC * NS
PER_W = N_TOTAL // NW
CHUNK = 10240
N_CHUNKS = PER_W // CHUNK

_mesh = plsc.VectorSubcoreMesh(core_axis_name="c", subcore_axis_name="s")


@functools.partial(
    pl.kernel,
    out_type=jax.ShapeDtypeStruct((N_TOTAL, EMBED), jnp.float32),
    mesh=_mesh,
    scratch_types=[
        pltpu.VMEM((CHUNK,), jnp.int32),
        pltpu.VMEM((CHUNK, EMBED), jnp.float32),
        pltpu.SemaphoreType.DMA,
    ],
)
def _gather_kernel(idx_hbm, w_hbm, out_hbm, idx_v, rows_v, sem):
    wid = lax.axis_index("s") * NC + lax.axis_index("c")
    base = wid * PER_W
    for c in range(N_CHUNKS):
        off = base + c * CHUNK
        pltpu.sync_copy(idx_hbm.at[pl.ds(off, CHUNK)], idx_v)
        pltpu.async_copy(w_hbm.at[idx_v], rows_v, sem).wait()
        pltpu.sync_copy(rows_v, out_hbm.at[pl.ds(off, CHUNK)])


def kernel(x, W):
    xf = x.reshape(N_TOTAL).astype(jnp.int32)
    out = _gather_kernel(xf, W)
    return out.reshape(BATCH, HIST, EMBED)


# speedup vs baseline: 15.9024x; 15.9024x over previous
import functools

import jax
import jax.numpy as jnp
from jax import lax
from jax.experimental import pallas as pl
from jax.experimental.pallas import tpu as pltpu
from jax.experimental.pallas import tpu_sc as plsc

BATCH = 16384
HIST = 200
EMBED = 2
N_TOTAL = BATCH * HIST
NC, NS = 2, 16
NW = NC * NS
PER_W = N_TOTAL // NW
CHUNK = 10240
N_CHUNKS = PER_W // CHUNK

_mesh = plsc.VectorSubcoreMesh(core_axis_name="c", subcore_axis_name="s")


@functools.partial(
    pl.kernel,
    out_type=jax.ShapeDtypeStruct((N_TOTAL, EMBED), jnp.float32),
    mesh=_mesh,
    scratch_types=[
        pltpu.VMEM((CHUNK,), jnp.int32),
        pltpu.VMEM((CHUNK, EMBED), jnp.float32),
        pltpu.SemaphoreType.DMA,
    ],
    compiler_params=pltpu.CompilerParams(use_tc_tiling_on_sc=False),
)
def _gather_kernel(idx_hbm, w_hbm, out_hbm, idx_v, rows_v, sem):
    wid = lax.axis_index("s") * NC + lax.axis_index("c")
    base = wid * PER_W
    for c in range(N_CHUNKS):
        off = base + c * CHUNK
        pltpu.sync_copy(idx_hbm.at[pl.ds(off, CHUNK)], idx_v)
        pltpu.async_copy(w_hbm.at[idx_v], rows_v, sem).wait()
        pltpu.sync_copy(rows_v, out_hbm.at[pl.ds(off, CHUNK)])


def kernel(x, W):
    xf = x.reshape(N_TOTAL).astype(jnp.int32)
    out = _gather_kernel(xf, W)
    return out.reshape(BATCH, HIST, EMBED)


# speedup vs baseline: 121.7407x; 7.6555x over previous
import functools

import jax
import jax.numpy as jnp
from jax import lax
from jax.experimental import pallas as pl
from jax.experimental.pallas import tpu as pltpu
from jax.experimental.pallas import tpu_sc as plsc

BATCH = 16384
HIST = 200
EMBED = 2
N_TOTAL = BATCH * HIST
NC, NS = 2, 16
NW = NC * NS
PER_W = N_TOTAL // NW
CHUNK = 10240
N_CHUNKS = PER_W // CHUNK

_mesh = plsc.VectorSubcoreMesh(core_axis_name="c", subcore_axis_name="s")


@functools.partial(
    pl.kernel,
    out_type=(
        jax.ShapeDtypeStruct((N_TOTAL,), jnp.float32),
        jax.ShapeDtypeStruct((N_TOTAL,), jnp.float32),
    ),
    mesh=_mesh,
    scratch_types=[
        pltpu.VMEM((CHUNK,), jnp.int32),
        pltpu.VMEM((CHUNK,), jnp.float32),
        pltpu.VMEM((CHUNK,), jnp.float32),
        pltpu.SemaphoreType.DMA,
        pltpu.SemaphoreType.DMA,
    ],
    compiler_params=pltpu.CompilerParams(use_tc_tiling_on_sc=False),
)
def _gather_kernel(idx_hbm, w0_hbm, w1_hbm, o0_hbm, o1_hbm,
                   idx_v, g0_v, g1_v, sem0, sem1):
    wid = lax.axis_index("s") * NC + lax.axis_index("c")
    base = wid * PER_W
    for c in range(N_CHUNKS):
        off = base + c * CHUNK
        pltpu.sync_copy(idx_hbm.at[pl.ds(off, CHUNK)], idx_v)
        cp0 = pltpu.async_copy(w0_hbm.at[idx_v], g0_v, sem0)
        cp1 = pltpu.async_copy(w1_hbm.at[idx_v], g1_v, sem1)
        cp0.wait()
        cp1.wait()
        pltpu.sync_copy(g0_v, o0_hbm.at[pl.ds(off, CHUNK)])
        pltpu.sync_copy(g1_v, o1_hbm.at[pl.ds(off, CHUNK)])


def kernel(x, W):
    xf = x.reshape(N_TOTAL).astype(jnp.int32)
    w0 = W[:, 0]
    w1 = W[:, 1]
    o0, o1 = _gather_kernel(xf, w0, w1)
    return jnp.stack(
        [o0.reshape(BATCH, HIST), o1.reshape(BATCH, HIST)], axis=-1)
